# baseline (speedup 1.0000x reference)
import functools

import jax
import jax.numpy as jnp
from jax import lax
from jax.experimental import pallas as pl
from jax.experimental.pallas import tpu as pltpu
from jax.experimental.pallas import tpu_sc as plsc

NU = 10000
D = 128
DW = 136
B = 4096
E = 320000

NC = 2
NS = 16
L = 16

NCH_UI = 156
EXTRA_BASE = 2496 * 128
NCH_UU0 = 128
NCH_UU1 = 28
UU1_BASE = 2048 * 128

RSZ = 10016

OFF_UU = 4160
OFF_UU2 = 8256
ACC_ROWS = 12416
TRASH_U = 4096
TRASH_I = 8192


def _sc_mesh():
    return plsc.VectorSubcoreMesh(
        core_axis_name="c", subcore_axis_name="s", num_cores=NC, num_subcores=NS
    )


def _sc_body(ua, ia, ui_ei, uu_ei, bu, bp, bn, pos,
             initu, initi, zrows,
             o_uego, o_uui, o_uuu1, o_uuu2, o_ip, o_in, o_isp, o_isn,
             ACC, RSTGU, RSTGI, rows, ebuf, tbuf, fslot, fsrc, gbuf, sbufF,
             bidx, posb, semA, semB):
    c = lax.axis_index("c")
    s = lax.axis_index("s")
    i32 = jnp.int32
    lanes = lax.iota(i32, L)

    def add_off(buf, off, n):
        for j in range(n // L):
            buf[pl.ds(j * L, L)] = buf[pl.ds(j * L, L)] + off

    pltpu.sync_copy(zrows, ACC.at[pl.ds(s * 776, 776)])

    @pl.when(s == 0)
    def _init_user_remap():
        pltpu.sync_copy(initu, RSTGU)

    @pl.when(jnp.logical_and(c == 1, s == 1))
    def _init_item_remap():
        pltpu.sync_copy(initi, RSTGI)

    plsc.subcore_barrier()

    def build(src_hbm, rstg, pos_base):
        def body(g, _):
            pltpu.sync_copy(src_hbm.at[pl.ds(g * 64, 64)], bidx)
            pltpu.sync_copy(pos.at[pl.ds(pos_base + g * 64, 64)], posb)
            pltpu.sync_copy(posb, rstg.at[bidx])
            return 0
        lax.fori_loop(0, B // 64, body, 0)

    @pl.when(s == 0)
    def _build_user_remap():
        build(bu, RSTGU, 0)

    @pl.when(jnp.logical_and(c == 1, s == 1))
    def _build_item_remap():
        build(bp, RSTGI, 0)
        build(bn, RSTGI, B)

    plsc.subcore_barrier()

    def fire(table):
        pltpu.async_copy(table.at[gbuf], rows, semA).wait()
        pltpu.async_copy(rows, ACC.at[sbufF], semB, add=True).wait()

    def edge_pass(edges, drow, srow, table, rstg, bound, soff, trash_base,
                  base, nch, nextra):
        def body(g, fc):
            off = jnp.where(g < nch, base + g * 128, EXTRA_BASE + s * 128)
            pltpu.sync_copy(edges.at[:, pl.ds(off, 128)], ebuf)
            pltpu.sync_copy(rstg.at[ebuf.at[drow]], tbuf)
            fcr = fc
            for j in range(8):
                sl = tbuf[pl.ds(j * L, L)]
                sv = ebuf[srow, pl.ds(j * L, L)]
                m = sl < bound
                if soff is not None:
                    sl = sl + soff
                plsc.store_compressed(fslot.at[pl.ds(fcr, L)], sl, mask=m)
                plsc.store_compressed(fsrc.at[pl.ds(fcr, L)], sv, mask=m)
                fcr = fcr + jnp.sum(m.astype(i32))

            @pl.when(fcr >= 128)
            def _fire_full():
                for j in range(8):
                    sbufF[pl.ds(j * L, L)] = fslot[pl.ds(j * L, L)]
                    gbuf[pl.ds(j * L, L)] = fsrc[pl.ds(j * L, L)]
                fire(table)
                for j in range(8):
                    fslot[pl.ds(j * L, L)] = fslot[pl.ds(128 + j * L, L)]
                    fsrc[pl.ds(j * L, L)] = fsrc[pl.ds(128 + j * L, L)]
            return jnp.where(fcr >= 128, fcr - 128, fcr)

        nch_t = nch + jnp.where(s < nextra, 1, 0)
        fc = lax.fori_loop(0, nch_t, body, jnp.int32(0))

        @pl.when(fc > 0)
        def _drain():
            for j in range(8):
                idxv = j * L + lanes
                m2 = idxv < fc
                tr = trash_base + (idxv & 63)
                sbufF[pl.ds(j * L, L)] = jnp.where(
                    m2, fslot[pl.ds(j * L, L)], tr)
                gbuf[pl.ds(j * L, L)] = jnp.where(
                    m2, fsrc[pl.ds(j * L, L)], 0)
            fire(table)

    @pl.when(c == 0)
    def _ui_user_pass():
        edge_pass(ui_ei, 0, 1, ia, RSTGU, TRASH_U, None, TRASH_U,
                  s * (NCH_UI * 128), NCH_UI, 4)

    @pl.when(c == 1)
    def _ui_item_pass():
        edge_pass(ui_ei, 1, 0, ua, RSTGI, TRASH_I, None, TRASH_I,
                  s * (NCH_UI * 128), NCH_UI, 4)

    uu_soff = jnp.where(c == 0, OFF_UU, OFF_UU2).astype(i32)
    uu_base = jnp.where(c == 0, s * (NCH_UU0 * 128),
                        UU1_BASE + s * (NCH_UU1 * 128))
    uu_nch = jnp.where(c == 0, NCH_UU0, NCH_UU1)
    uu_nextra = jnp.where(c == 0, 4, 0)
    edge_pass(uu_ei, 0, 1, ua, RSTGU, TRASH_U, uu_soff, uu_soff + TRASH_U,
              uu_base, uu_nch, uu_nextra)

    plsc.subcore_barrier()

    r64 = rows.at[pl.ds(0, 64)]
    s64 = tbuf.at[pl.ds(0, 64)]

    def hbm_gather(src_hbm, o, table, out):
        pltpu.sync_copy(src_hbm.at[pl.ds(o, 64)], bidx)
        pltpu.async_copy(table.at[bidx], r64, semA).wait()
        pltpu.sync_copy(r64, out.at[pl.ds(o, 64)])

    def acc_gather(rstg, o, off, out):
        pltpu.sync_copy(rstg.at[bidx], s64)
        if off is not None:
            add_off(tbuf, off, 64)
        pltpu.sync_copy(ACC.at[s64], r64)
        pltpu.sync_copy(r64, out.at[pl.ds(o, 64)])

    @pl.when(c == 0)
    def _final_user_side():
        def fin(k, _):
            o = s * 256 + k * 64
            hbm_gather(bu, o, ua, o_uego)
            acc_gather(RSTGU, o, None, o_uui)
            add_off(tbuf, OFF_UU, 64)
            pltpu.sync_copy(ACC.at[s64], r64)
            pltpu.sync_copy(r64, o_uuu1.at[pl.ds(o, 64)])
            return 0
        lax.fori_loop(0, 4, fin, 0)

    @pl.when(c == 1)
    def _final_item_side():
        def fin(k, _):
            o = s * 256 + k * 64
            pltpu.sync_copy(bu.at[pl.ds(o, 64)], bidx)
            acc_gather(RSTGU, o, OFF_UU2, o_uuu2)
            hbm_gather(bp, o, ia, o_ip)
            acc_gather(RSTGI, o, None, o_isp)
            hbm_gather(bn, o, ia, o_in)
            acc_gather(RSTGI, o, None, o_isn)
            return 0
        lax.fori_loop(0, 4, fin, 0)


_sc_call = functools.partial(
    pl.kernel,
    out_type=[jax.ShapeDtypeStruct((B, DW), jnp.float32)] * 8,
    mesh=_sc_mesh(),
    compiler_params=pltpu.CompilerParams(
        needs_layout_passes=False, use_tc_tiling_on_sc=False),
    scratch_types=[
        pltpu.VMEM_SHARED((ACC_ROWS, DW), jnp.float32),
        pltpu.VMEM_SHARED((RSZ,), jnp.int32),
        pltpu.VMEM_SHARED((RSZ,), jnp.int32),
        pltpu.VMEM((128, DW), jnp.float32),
        pltpu.VMEM((2, 128), jnp.int32),
        pltpu.VMEM((128,), jnp.int32),
        pltpu.VMEM((256,), jnp.int32),
        pltpu.VMEM((256,), jnp.int32),
        pltpu.VMEM((128,), jnp.int32),
        pltpu.VMEM((128,), jnp.int32),
        pltpu.VMEM((64,), jnp.int32),
        pltpu.VMEM((64,), jnp.int32),
        pltpu.SemaphoreType.DMA,
        pltpu.SemaphoreType.DMA,
    ],
)(_sc_body)


def _tc_body(ue, uui, du, uu1, uu2, duu1, duu2, ipr, inr, isp, dp, isn, dn,
             W0, W1, W2, bm, out):
    f32 = jnp.float32
    ue_ = ue[...]
    un = uui[...] / jnp.maximum(du[...], 1.0)
    uu = (uu1[...] + uu2[...]) / jnp.maximum(duu1[...] + duu2[...], 1.0)
    h = (
        jnp.dot(ue_, W0[...], preferred_element_type=f32)
        + jnp.dot(un, W1[...], preferred_element_type=f32)
        + jnp.dot(uu, W2[...], preferred_element_type=f32)
        + bm[...]
    )
    u = jnp.tanh(h)
    ipr_ = ipr[...]
    inr_ = inr[...]
    p = 0.5 * (isp[...] / jnp.maximum(dp[...], 1.0)) + 0.5 * ipr_
    n = 0.5 * (isn[...] / jnp.maximum(dn[...], 1.0)) + 0.5 * inr_
    diff = jnp.sum(u * n, axis=-1) - jnp.sum(u * p, axis=-1)
    cf = jnp.mean(jnp.maximum(diff, 0.0) + jnp.log1p(jnp.exp(-jnp.abs(diff))))
    reg = 0.5 * jnp.mean(
        jnp.sum(ue_ * ue_, axis=-1)
        + jnp.sum(ipr_ * ipr_, axis=-1)
        + jnp.sum(inr_ * inr_, axis=-1)
    )
    out[...] = jnp.reshape(1.0 * cf + 1e-4 * reg, (1, 1))


def kernel(user_emb, item_emb, W_map, b_map, ui_edge_index, uu_edge_index,
           batch_user_pos_neg):
    f32 = jnp.float32
    i32 = jnp.int32

    ones_col = jnp.ones((NU, 1), f32)
    pad_cols = jnp.zeros((NU, DW - D - 1), f32)
    ua = jnp.concatenate([user_emb, ones_col, pad_cols], axis=1)
    ia = jnp.concatenate([item_emb, ones_col, pad_cols], axis=1)

    ui_ei = ui_edge_index.astype(i32)
    uu_ei = uu_edge_index.astype(i32)

    bu = batch_user_pos_neg[:, 0].astype(i32)
    bp = batch_user_pos_neg[:, 1].astype(i32)
    bn = batch_user_pos_neg[:, 2].astype(i32)

    pos = jnp.arange(2 * B, dtype=i32)
    ar = jnp.arange(RSZ, dtype=i32)
    initu = TRASH_U + (ar & 63)
    initi = TRASH_I + (ar & 63)
    zrows = jnp.zeros((776, DW), f32)

    (uego_a, uui_a, uuu1_a, uuu2_a, ip_a, in_a, isp_a, isn_a) = _sc_call(
        ua, ia, ui_ei, uu_ei, bu, bp, bn, pos, initu, initi, zrows)

    def split(a):
        return a[:, :D], a[:, D:D + 1]

    ue_, _ = split(uego_a)
    uui_, du = split(uui_a)
    uu1_, duu1 = split(uuu1_a)
    uu2_, duu2 = split(uuu2_a)
    ipr_, _ = split(ip_a)
    inr_, _ = split(in_a)
    isp_, dp = split(isp_a)
    isn_, dn = split(isn_a)

    W0 = W_map[:D]
    W1 = W_map[D:2 * D]
    W2 = W_map[2 * D:]
    bm = b_map.reshape(1, D)

    out = pl.pallas_call(
        _tc_body,
        out_shape=jax.ShapeDtypeStruct((1, 1), f32),
    )(ue_, uui_, du, uu1_, uu2_, duu1, duu2, ipr_, inr_, isp_, dp, isn_, dn,
      W0, W1, W2, bm)
    return out[0, 0]


# speedup vs baseline: 1.0231x; 1.0231x over previous
import functools

import jax
import jax.numpy as jnp
from jax import lax
from jax.experimental import pallas as pl
from jax.experimental.pallas import tpu as pltpu
from jax.experimental.pallas import tpu_sc as plsc

NU = 10000
D = 128
DW = 136
B = 4096

NC = 2
NS = 16
L = 16

E = 320000
NCH_UI = 250
NCH_UU = 125

RSZ = 10016

OFF_UU = 4160
OFF_UU2 = 8256
ACC_ROWS = 12416
TRASH_U = 4096
TRASH_I = 8192


def _sc_mesh():
    return plsc.VectorSubcoreMesh(
        core_axis_name="c", subcore_axis_name="s", num_cores=NC, num_subcores=NS
    )


def _sc_body(ua, ia, ui_ei, uu_ei, bu, bp, bn, pos,
             initu, initi, zrows,
             o_uego, o_uui, o_uuu1, o_uuu2, o_ip, o_in, o_isp, o_isn,
             ACC, RSTGU, RSTGI, rows0, rows1, ebuf0, ebuf1, sbuf0, sbuf1,
             bidx, posb, semA0, semA1, semB0, semB1, semE0, semE1,
             semX0, semX1):
    c = lax.axis_index("c")
    s = lax.axis_index("s")
    rows = (rows0, rows1)
    ebuf = (ebuf0, ebuf1)
    sbuf = (sbuf0, sbuf1)
    semA = (semA0, semA1)
    semB = (semB0, semB1)
    semE = (semE0, semE1)
    semX = (semX0, semX1)

    def add_off(buf, off, n):
        for j in range(n // L):
            buf[pl.ds(j * L, L)] = buf[pl.ds(j * L, L)] + off

    pltpu.sync_copy(zrows, ACC.at[pl.ds(s * 776, 776)])

    @pl.when(s == 0)
    def _init_user_remap():
        pltpu.sync_copy(initu, RSTGU)

    @pl.when(jnp.logical_and(c == 1, s == 1))
    def _init_item_remap():
        pltpu.sync_copy(initi, RSTGI)

    plsc.subcore_barrier()

    def build(src_hbm, rstg, pos_base):
        def body(g, _):
            pltpu.sync_copy(src_hbm.at[pl.ds(g * 64, 64)], bidx)
            pltpu.sync_copy(pos.at[pl.ds(pos_base + g * 64, 64)], posb)
            pltpu.sync_copy(posb, rstg.at[bidx])
            return 0
        lax.fori_loop(0, B // 64, body, 0)

    @pl.when(s == 0)
    def _build_user_remap():
        build(bu, RSTGU, 0)

    @pl.when(jnp.logical_and(c == 1, s == 1))
    def _build_item_remap():
        build(bp, RSTGI, 0)
        build(bn, RSTGI, B)

    plsc.subcore_barrier()

    def edge_pass(edges, table, rstg, base, nch, slot_off):
        def off_(g):
            return base + g * 80

        def Es(b, g):
            pltpu.async_copy(edges.at[:, pl.ds(off_(g), 80)], ebuf[b],
                             semE[b])

        def Ew(b, g):
            pltpu.make_async_copy(edges.at[:, pl.ds(off_(g), 80)], ebuf[b],
                                  semE[b]).wait()

        def Xs(b):
            pltpu.async_copy(rstg.at[ebuf[b].at[0]], sbuf[b], semX[b])

        def Xw(b):
            pltpu.make_async_copy(rstg.at[ebuf[b].at[0]], sbuf[b],
                                  semX[b]).wait()
            if slot_off is not None:
                add_off(sbuf[b], slot_off, 80)

        def Gs(b):
            pltpu.async_copy(table.at[ebuf[b].at[1]], rows[b], semA[b])

        def Gw(b):
            pltpu.make_async_copy(table.at[ebuf[b].at[1]], rows[b],
                                  semA[b]).wait()

        def Ss(b):
            pltpu.async_copy(rows[b], ACC.at[sbuf[b]], semB[b], add=True)

        def Sw(b):
            pltpu.make_async_copy(rows[b], ACC.at[sbuf[b]], semB[b]).wait()

        npairs = nch // 2
        Es(0, 0)
        Ew(0, 0)
        Xs(0)
        Gs(0)

        def body(h, _):
            e = 2 * h
            o = 2 * h + 1
            Es(1, o)
            Xw(0)
            Gw(0)

            @pl.when(h > 0)
            def _swo():
                Sw(1)
            Ss(0)
            Ew(1, o)
            Xs(1)
            Gs(1)

            @pl.when(h < npairs - 1)
            def _prime_e():
                Es(0, e + 2)
            Xw(1)
            Gw(1)
            Sw(0)
            Ss(1)

            @pl.when(h < npairs - 1)
            def _prime_rest():
                Ew(0, e + 2)
                Xs(0)
                Gs(0)
            return 0
        lax.fori_loop(0, npairs, body, 0)
        Sw(1)

        if nch % 2 == 1:
            g = nch - 1
            Es(0, g)
            Ew(0, g)
            Xs(0)
            Gs(0)
            Xw(0)
            Gw(0)
            Ss(0)
            Sw(0)

    @pl.when(c == 0)
    def _ui_user_pass():
        edge_pass(ui_ei, ia, RSTGU, s * (NCH_UI * 80), NCH_UI, None)

    @pl.when(c == 1)
    def _ui_item_pass():
        edge_pass(ui_ei, ua, RSTGI, s * (NCH_UI * 80), NCH_UI, None)

    uu_off = jnp.where(c == 0, OFF_UU, OFF_UU2).astype(jnp.int32)
    edge_pass(uu_ei, ua, RSTGU, c * (E // NC) + s * (NCH_UU * 80), NCH_UU,
              uu_off)

    plsc.subcore_barrier()

    r64 = rows0.at[pl.ds(0, 64)]
    s64 = sbuf0.at[pl.ds(0, 64)]

    def hbm_gather(src_hbm, o, table, out):
        pltpu.sync_copy(src_hbm.at[pl.ds(o, 64)], bidx)
        pltpu.async_copy(table.at[bidx], r64, semA0).wait()
        pltpu.sync_copy(r64, out.at[pl.ds(o, 64)])

    def acc_gather(rstg, o, off, out):
        pltpu.sync_copy(rstg.at[bidx], s64)
        if off is not None:
            add_off(sbuf0, off, 64)
        pltpu.sync_copy(ACC.at[s64], r64)
        pltpu.sync_copy(r64, out.at[pl.ds(o, 64)])

    @pl.when(c == 0)
    def _final_user_side():
        def fin(k, _):
            o = s * 256 + k * 64
            hbm_gather(bu, o, ua, o_uego)
            acc_gather(RSTGU, o, None, o_uui)
            add_off(sbuf0, OFF_UU, 64)
            pltpu.sync_copy(ACC.at[s64], r64)
            pltpu.sync_copy(r64, o_uuu1.at[pl.ds(o, 64)])
            return 0
        lax.fori_loop(0, 4, fin, 0)

    @pl.when(c == 1)
    def _final_item_side():
        def fin(k, _):
            o = s * 256 + k * 64
            pltpu.sync_copy(bu.at[pl.ds(o, 64)], bidx)
            acc_gather(RSTGU, o, OFF_UU2, o_uuu2)
            hbm_gather(bp, o, ia, o_ip)
            acc_gather(RSTGI, o, None, o_isp)
            hbm_gather(bn, o, ia, o_in)
            acc_gather(RSTGI, o, None, o_isn)
            return 0
        lax.fori_loop(0, 4, fin, 0)


_sc_call = functools.partial(
    pl.kernel,
    out_type=[jax.ShapeDtypeStruct((B, DW), jnp.float32)] * 8,
    mesh=_sc_mesh(),
    compiler_params=pltpu.CompilerParams(
        needs_layout_passes=False, use_tc_tiling_on_sc=False),
    scratch_types=[
        pltpu.VMEM_SHARED((ACC_ROWS, DW), jnp.float32),
        pltpu.VMEM_SHARED((RSZ,), jnp.int32),
        pltpu.VMEM_SHARED((RSZ,), jnp.int32),
        pltpu.VMEM((80, DW), jnp.float32),
        pltpu.VMEM((80, DW), jnp.float32),
        pltpu.VMEM((2, 80), jnp.int32),
        pltpu.VMEM((2, 80), jnp.int32),
        pltpu.VMEM((80,), jnp.int32),
        pltpu.VMEM((80,), jnp.int32),
        pltpu.VMEM((64,), jnp.int32),
        pltpu.VMEM((64,), jnp.int32),
        pltpu.SemaphoreType.DMA,
        pltpu.SemaphoreType.DMA,
        pltpu.SemaphoreType.DMA,
        pltpu.SemaphoreType.DMA,
        pltpu.SemaphoreType.DMA,
        pltpu.SemaphoreType.DMA,
        pltpu.SemaphoreType.DMA,
        pltpu.SemaphoreType.DMA,
    ],
)(_sc_body)


def _tc_body(ue, uui, du, uu1, uu2, duu1, duu2, ipr, inr, isp, dp, isn, dn,
             W0, W1, W2, bm, out):
    f32 = jnp.float32
    ue_ = ue[...]
    un = uui[...] / jnp.maximum(du[...], 1.0)
    uu = (uu1[...] + uu2[...]) / jnp.maximum(duu1[...] + duu2[...], 1.0)
    h = (
        jnp.dot(ue_, W0[...], preferred_element_type=f32)
        + jnp.dot(un, W1[...], preferred_element_type=f32)
        + jnp.dot(uu, W2[...], preferred_element_type=f32)
        + bm[...]
    )
    u = jnp.tanh(h)
    ipr_ = ipr[...]
    inr_ = inr[...]
    p = 0.5 * (isp[...] / jnp.maximum(dp[...], 1.0)) + 0.5 * ipr_
    n = 0.5 * (isn[...] / jnp.maximum(dn[...], 1.0)) + 0.5 * inr_
    diff = jnp.sum(u * n, axis=-1) - jnp.sum(u * p, axis=-1)
    cf = jnp.mean(jnp.maximum(diff, 0.0) + jnp.log1p(jnp.exp(-jnp.abs(diff))))
    reg = 0.5 * jnp.mean(
        jnp.sum(ue_ * ue_, axis=-1)
        + jnp.sum(ipr_ * ipr_, axis=-1)
        + jnp.sum(inr_ * inr_, axis=-1)
    )
    out[...] = jnp.reshape(1.0 * cf + 1e-4 * reg, (1, 1))


def kernel(user_emb, item_emb, W_map, b_map, ui_edge_index, uu_edge_index,
           batch_user_pos_neg):
    f32 = jnp.float32
    i32 = jnp.int32

    ones_col = jnp.ones((NU, 1), f32)
    pad_cols = jnp.zeros((NU, DW - D - 1), f32)
    ua = jnp.concatenate([user_emb, ones_col, pad_cols], axis=1)
    ia = jnp.concatenate([item_emb, ones_col, pad_cols], axis=1)

    ui_ei = ui_edge_index.astype(i32)
    uu_ei = uu_edge_index.astype(i32)

    bu = batch_user_pos_neg[:, 0].astype(i32)
    bp = batch_user_pos_neg[:, 1].astype(i32)
    bn = batch_user_pos_neg[:, 2].astype(i32)

    pos = jnp.arange(2 * B, dtype=i32)
    ar = jnp.arange(RSZ, dtype=i32)
    initu = TRASH_U + (ar & 63)
    initi = TRASH_I + (ar & 63)
    zrows = jnp.zeros((776, DW), f32)

    (uego_a, uui_a, uuu1_a, uuu2_a, ip_a, in_a, isp_a, isn_a) = _sc_call(
        ua, ia, ui_ei, uu_ei, bu, bp, bn, pos, initu, initi, zrows)

    def split(a):
        return a[:, :D], a[:, D:D + 1]

    ue_, _ = split(uego_a)
    uui_, du = split(uui_a)
    uu1_, duu1 = split(uuu1_a)
    uu2_, duu2 = split(uuu2_a)
    ipr_, _ = split(ip_a)
    inr_, _ = split(in_a)
    isp_, dp = split(isp_a)
    isn_, dn = split(isn_a)

    W0 = W_map[:D]
    W1 = W_map[D:2 * D]
    W2 = W_map[2 * D:]
    bm = b_map.reshape(1, D)

    out = pl.pallas_call(
        _tc_body,
        out_shape=jax.ShapeDtypeStruct((1, 1), f32),
    )(ue_, uui_, du, uu1_, uu2_, duu1, duu2, ipr_, inr_, isp_, dp, isn_, dn,
      W0, W1, W2, bm)
    return out[0, 0]


# speedup vs baseline: 1.0610x; 1.0370x over previous
import functools

import jax
import jax.numpy as jnp
from jax import lax
from jax.experimental import pallas as pl
from jax.experimental.pallas import tpu as pltpu
from jax.experimental.pallas import tpu_sc as plsc

NU = 10000
D = 128
DW = 136
B = 4096
E = 320000

NC = 2
NS = 16
L = 16
F = 96

NCH_UI = 156
EXTRA_BASE = 2496 * 128
NCH_UU0 = 128
NCH_UU1 = 28
UU1_BASE = 2048 * 128

RSZ = 10016

OFF_UU = 4160
OFF_UU2 = 8256
ACC_ROWS = 12416
TRASH_U = 4096
TRASH_I = 8192


def _sc_mesh():
    return plsc.VectorSubcoreMesh(
        core_axis_name="c", subcore_axis_name="s", num_cores=NC, num_subcores=NS
    )


def _sc_body(ua, ia, ui_ei, uu_ei, bu, bp, bn, pos,
             initu, initi, zrows,
             o_uego, o_uui, o_uuu1, o_uuu2, o_ip, o_in, o_isp, o_isn,
             ACC, RSTGU, RSTGI, RL, rows, ebuf, tbuf, fslot, fsrc, gbuf,
             sbufF, bidx, semA, semB):
    c = lax.axis_index("c")
    s = lax.axis_index("s")
    i32 = jnp.int32
    lanes = lax.iota(i32, L)

    def add_off(buf, off, n):
        for j in range(n // L):
            buf[pl.ds(j * L, L)] = buf[pl.ds(j * L, L)] + off

    pltpu.sync_copy(zrows, ACC.at[pl.ds(s * 776, 776)])

    @pl.when(s == 0)
    def _init_user_remap():
        pltpu.sync_copy(initu, RSTGU)

    @pl.when(jnp.logical_and(c == 1, s == 1))
    def _init_item_remap():
        pltpu.sync_copy(initi, RSTGI)

    plsc.subcore_barrier()

    def build(src_hbm, rstg, pos_base):
        def body(g, _):
            pltpu.sync_copy(src_hbm.at[pl.ds(g * 64, 64)], bidx)
            pltpu.sync_copy(pos.at[pl.ds(pos_base + g * 64, 64)],
                            sbufF.at[pl.ds(0, 64)])
            pltpu.sync_copy(sbufF.at[pl.ds(0, 64)], rstg.at[bidx])
            return 0
        lax.fori_loop(0, B // 64, body, 0)

    @pl.when(s == 0)
    def _build_user_remap():
        build(bu, RSTGU, 0)

    @pl.when(jnp.logical_and(c == 1, s == 1))
    def _build_item_remap():
        build(bp, RSTGI, 0)
        build(bn, RSTGI, B)

    plsc.subcore_barrier()

    @pl.when(c == 0)
    def _fetch_user_remap():
        pltpu.sync_copy(RSTGU, RL)

    @pl.when(c == 1)
    def _fetch_item_remap():
        pltpu.sync_copy(RSTGI, RL)

    def fire(table):
        pltpu.async_copy(table.at[gbuf], rows, semA).wait()
        pltpu.async_copy(rows, ACC.at[sbufF], semB, add=True).wait()

    def fifo_to_bufs_and_fire(table):
        for j in range(F // L):
            sbufF[pl.ds(j * L, L)] = fslot[pl.ds(j * L, L)]
            gbuf[pl.ds(j * L, L)] = fsrc[pl.ds(j * L, L)]
        fire(table)
        for j in range(8):
            fslot[pl.ds(j * L, L)] = fslot[pl.ds(F + j * L, L)]
            fsrc[pl.ds(j * L, L)] = fsrc[pl.ds(F + j * L, L)]

    def edge_pass(edges, drow, srow, table, local_rl, rstg, bound, soff,
                  trash_base, base, nch, nextra):
        def body(g, fc):
            off = jnp.where(g < nch, base + g * 128, EXTRA_BASE + s * 128)
            pltpu.sync_copy(edges.at[:, pl.ds(off, 128)], ebuf)
            if not local_rl:
                pltpu.sync_copy(rstg.at[ebuf.at[drow]], tbuf)
            fcr = fc
            for j in range(8):
                dv = ebuf[drow, pl.ds(j * L, L)]
                if local_rl:
                    sl = plsc.load_gather(RL, [dv])
                else:
                    sl = tbuf[pl.ds(j * L, L)]
                sv = ebuf[srow, pl.ds(j * L, L)]
                m = sl < bound
                if soff is not None:
                    sl = sl + soff
                plsc.store_compressed(fslot.at[pl.ds(fcr, L)], sl, mask=m)
                plsc.store_compressed(fsrc.at[pl.ds(fcr, L)], sv, mask=m)
                fcr = fcr + jnp.sum(m.astype(i32))

            @pl.when(fcr >= F)
            def _fire1():
                fifo_to_bufs_and_fire(table)

            @pl.when(fcr >= 2 * F)
            def _fire2():
                fifo_to_bufs_and_fire(table)
            nfired = (fcr >= F).astype(i32) + (fcr >= 2 * F).astype(i32)
            return fcr - F * nfired

        nch_t = nch + jnp.where(s < nextra, 1, 0)
        fc = lax.fori_loop(0, nch_t, body, jnp.int32(0))

        @pl.when(fc > 0)
        def _drain():
            for j in range(F // L):
                idxv = j * L + lanes
                m2 = idxv < fc
                tr = trash_base + (idxv & 63)
                sbufF[pl.ds(j * L, L)] = jnp.where(
                    m2, fslot[pl.ds(j * L, L)], tr)
                gbuf[pl.ds(j * L, L)] = jnp.where(
                    m2, fsrc[pl.ds(j * L, L)], 0)
            fire(table)

    @pl.when(c == 0)
    def _sc0_passes():
        edge_pass(ui_ei, 0, 1, ia, True, RSTGU, TRASH_U, None, TRASH_U,
                  s * (NCH_UI * 128), NCH_UI, 4)
        edge_pass(uu_ei, 0, 1, ua, True, RSTGU, TRASH_U, OFF_UU,
                  OFF_UU + TRASH_U, s * (NCH_UU0 * 128), NCH_UU0, 4)

    @pl.when(c == 1)
    def _sc1_passes():
        edge_pass(ui_ei, 1, 0, ua, True, RSTGI, TRASH_I, None, TRASH_I,
                  s * (NCH_UI * 128), NCH_UI, 4)
        edge_pass(uu_ei, 0, 1, ua, False, RSTGU, TRASH_U, OFF_UU2,
                  OFF_UU2 + TRASH_U, UU1_BASE + s * (NCH_UU1 * 128),
                  NCH_UU1, 0)

    plsc.subcore_barrier()

    r64 = rows.at[pl.ds(0, 64)]
    s64 = tbuf.at[pl.ds(0, 64)]

    def hbm_gather(src_hbm, o, table, out):
        pltpu.sync_copy(src_hbm.at[pl.ds(o, 64)], bidx)
        pltpu.async_copy(table.at[bidx], r64, semA).wait()
        pltpu.sync_copy(r64, out.at[pl.ds(o, 64)])

    def acc_gather(rstg, o, off, out):
        pltpu.sync_copy(rstg.at[bidx], s64)
        if off is not None:
            add_off(tbuf, off, 64)
        pltpu.sync_copy(ACC.at[s64], r64)
        pltpu.sync_copy(r64, out.at[pl.ds(o, 64)])

    @pl.when(c == 0)
    def _final_user_side():
        def fin(k, _):
            o = s * 256 + k * 64
            hbm_gather(bu, o, ua, o_uego)
            acc_gather(RSTGU, o, None, o_uui)
            add_off(tbuf, OFF_UU, 64)
            pltpu.sync_copy(ACC.at[s64], r64)
            pltpu.sync_copy(r64, o_uuu1.at[pl.ds(o, 64)])
            return 0
        lax.fori_loop(0, 4, fin, 0)

    @pl.when(c == 1)
    def _final_item_side():
        def fin(k, _):
            o = s * 256 + k * 64
            pltpu.sync_copy(bu.at[pl.ds(o, 64)], bidx)
            acc_gather(RSTGU, o, OFF_UU2, o_uuu2)
            hbm_gather(bp, o, ia, o_ip)
            acc_gather(RSTGI, o, None, o_isp)
            hbm_gather(bn, o, ia, o_in)
            acc_gather(RSTGI, o, None, o_isn)
            return 0
        lax.fori_loop(0, 4, fin, 0)


_sc_call = functools.partial(
    pl.kernel,
    out_type=[jax.ShapeDtypeStruct((B, DW), jnp.float32)] * 8,
    mesh=_sc_mesh(),
    compiler_params=pltpu.CompilerParams(
        needs_layout_passes=False, use_tc_tiling_on_sc=False),
    scratch_types=[
        pltpu.VMEM_SHARED((ACC_ROWS, DW), jnp.float32),
        pltpu.VMEM_SHARED((RSZ,), jnp.int32),
        pltpu.VMEM_SHARED((RSZ,), jnp.int32),
        pltpu.VMEM((RSZ,), jnp.int32),
        pltpu.VMEM((F, DW), jnp.float32),
        pltpu.VMEM((2, 128), jnp.int32),
        pltpu.VMEM((128,), jnp.int32),
        pltpu.VMEM((256,), jnp.int32),
        pltpu.VMEM((256,), jnp.int32),
        pltpu.VMEM((F,), jnp.int32),
        pltpu.VMEM((F,), jnp.int32),
        pltpu.VMEM((64,), jnp.int32),
        pltpu.SemaphoreType.DMA,
        pltpu.SemaphoreType.DMA,
    ],
)(_sc_body)


def _tc_body(ue, uui, du, uu1, uu2, duu1, duu2, ipr, inr, isp, dp, isn, dn,
             W0, W1, W2, bm, out):
    f32 = jnp.float32
    ue_ = ue[...]
    un = uui[...] / jnp.maximum(du[...], 1.0)
    uu = (uu1[...] + uu2[...]) / jnp.maximum(duu1[...] + duu2[...], 1.0)
    h = (
        jnp.dot(ue_, W0[...], preferred_element_type=f32)
        + jnp.dot(un, W1[...], preferred_element_type=f32)
        + jnp.dot(uu, W2[...], preferred_element_type=f32)
        + bm[...]
    )
    u = jnp.tanh(h)
    ipr_ = ipr[...]
    inr_ = inr[...]
    p = 0.5 * (isp[...] / jnp.maximum(dp[...], 1.0)) + 0.5 * ipr_
    n = 0.5 * (isn[...] / jnp.maximum(dn[...], 1.0)) + 0.5 * inr_
    diff = jnp.sum(u * n, axis=-1) - jnp.sum(u * p, axis=-1)
    cf = jnp.mean(jnp.maximum(diff, 0.0) + jnp.log1p(jnp.exp(-jnp.abs(diff))))
    reg = 0.5 * jnp.mean(
        jnp.sum(ue_ * ue_, axis=-1)
        + jnp.sum(ipr_ * ipr_, axis=-1)
        + jnp.sum(inr_ * inr_, axis=-1)
    )
    out[...] = jnp.reshape(1.0 * cf + 1e-4 * reg, (1, 1))


def kernel(user_emb, item_emb, W_map, b_map, ui_edge_index, uu_edge_index,
           batch_user_pos_neg):
    f32 = jnp.float32
    i32 = jnp.int32

    ones_col = jnp.ones((NU, 1), f32)
    pad_cols = jnp.zeros((NU, DW - D - 1), f32)
    ua = jnp.concatenate([user_emb, ones_col, pad_cols], axis=1)
    ia = jnp.concatenate([item_emb, ones_col, pad_cols], axis=1)

    ui_ei = ui_edge_index.astype(i32)
    uu_ei = uu_edge_index.astype(i32)

    bu = batch_user_pos_neg[:, 0].astype(i32)
    bp = batch_user_pos_neg[:, 1].astype(i32)
    bn = batch_user_pos_neg[:, 2].astype(i32)

    pos = jnp.arange(2 * B, dtype=i32)
    ar = jnp.arange(RSZ, dtype=i32)
    initu = TRASH_U + (ar & 63)
    initi = TRASH_I + (ar & 63)
    zrows = jnp.zeros((776, DW), f32)

    (uego_a, uui_a, uuu1_a, uuu2_a, ip_a, in_a, isp_a, isn_a) = _sc_call(
        ua, ia, ui_ei, uu_ei, bu, bp, bn, pos, initu, initi, zrows)

    def split(a):
        return a[:, :D], a[:, D:D + 1]

    ue_, _ = split(uego_a)
    uui_, du = split(uui_a)
    uu1_, duu1 = split(uuu1_a)
    uu2_, duu2 = split(uuu2_a)
    ipr_, _ = split(ip_a)
    inr_, _ = split(in_a)
    isp_, dp = split(isp_a)
    isn_, dn = split(isn_a)

    W0 = W_map[:D]
    W1 = W_map[D:2 * D]
    W2 = W_map[2 * D:]
    bm = b_map.reshape(1, D)

    out = pl.pallas_call(
        _tc_body,
        out_shape=jax.ShapeDtypeStruct((1, 1), f32),
    )(ue_, uui_, du, uu1_, uu2_, duu1, duu2, ipr_, inr_, isp_, dp, isn_, dn,
      W0, W1, W2, bm)
    return out[0, 0]


# speedup vs baseline: 1.1618x; 1.0951x over previous
import functools

import jax
import jax.numpy as jnp
from jax import lax
from jax.experimental import pallas as pl
from jax.experimental.pallas import tpu as pltpu
from jax.experimental.pallas import tpu_sc as plsc

NU = 10000
D = 128
DW = 136
B = 4096
E = 320000

NC = 2
NS = 16
L = 16
F = 96

NCH_UI = 156
EXTRA_BASE = 2496 * 128
NCH_UU0 = 128
NCH_UU1 = 28
UU1_BASE = 2048 * 128

RSZ = 10016

OFF_UU = 4160
OFF_UU2 = 8256
ACC_ROWS = 12416
TRASH_U = 4096
TRASH_I = 8192


def _sc_mesh():
    return plsc.VectorSubcoreMesh(
        core_axis_name="c", subcore_axis_name="s", num_cores=NC, num_subcores=NS
    )


def _sc_body(ua, ia, ui_ei, uu_ei, bu, bp, bn, pos,
             initu, initi, zrows,
             o_uego, o_uui, o_uuu1, o_uuu2, o_ip, o_in, o_isp, o_isn,
             ACC, RSTGU, RSTGI, RL, rows, ebuf, tbuf, fslot, fsrc, gbuf,
             sbufF, bidx, semA, semB):
    c = lax.axis_index("c")
    s = lax.axis_index("s")
    i32 = jnp.int32
    lanes = lax.iota(i32, L)

    def add_off(buf, off, n):
        for j in range(n // L):
            buf[pl.ds(j * L, L)] = buf[pl.ds(j * L, L)] + off

    pltpu.sync_copy(zrows, ACC.at[pl.ds(s * 776, 776)])

    @pl.when(s == 0)
    def _init_user_remap():
        pltpu.sync_copy(initu, RSTGU)

    @pl.when(jnp.logical_and(c == 1, s == 1))
    def _init_item_remap():
        pltpu.sync_copy(initi, RSTGI)

    plsc.subcore_barrier()

    def build(src_hbm, rstg, pos_base):
        def body(g, _):
            pltpu.sync_copy(src_hbm.at[pl.ds(g * 64, 64)], bidx)
            pltpu.sync_copy(pos.at[pl.ds(pos_base + g * 64, 64)],
                            sbufF.at[pl.ds(0, 64)])
            pltpu.sync_copy(sbufF.at[pl.ds(0, 64)], rstg.at[bidx])
            return 0
        lax.fori_loop(0, B // 64, body, 0)

    @pl.when(s == 0)
    def _build_user_remap():
        build(bu, RSTGU, 0)

    @pl.when(jnp.logical_and(c == 1, s == 1))
    def _build_item_remap():
        build(bp, RSTGI, 0)
        build(bn, RSTGI, B)

    plsc.subcore_barrier()

    @pl.when(c == 0)
    def _fetch_user_remap():
        pltpu.sync_copy(RSTGU, RL)

    @pl.when(c == 1)
    def _fetch_item_remap():
        pltpu.sync_copy(RSTGI, RL)

    def s_wait():
        pltpu.make_async_copy(rows, ACC.at[sbufF], semB).wait()

    def fire(table):
        pltpu.async_copy(table.at[gbuf], rows, semA).wait()
        pltpu.async_copy(rows, ACC.at[sbufF], semB, add=True)

    def fifo_to_bufs_and_fire(table):
        for j in range(F // L):
            sbufF[pl.ds(j * L, L)] = fslot[pl.ds(j * L, L)]
            gbuf[pl.ds(j * L, L)] = fsrc[pl.ds(j * L, L)]
        fire(table)
        for j in range(8):
            fslot[pl.ds(j * L, L)] = fslot[pl.ds(F + j * L, L)]
            fsrc[pl.ds(j * L, L)] = fsrc[pl.ds(F + j * L, L)]

    def edge_pass(edges, drow, srow, table, local_rl, rstg, bound, soff,
                  trash_base, base, nch, nextra):
        def body_inner(g, fc, fk):
            off = jnp.where(g < nch, base + g * 128, EXTRA_BASE + s * 128)
            pltpu.sync_copy(edges.at[:, pl.ds(off, 128)], ebuf)
            if not local_rl:
                pltpu.sync_copy(rstg.at[ebuf.at[drow]], tbuf)
            fcr = fc
            for j in range(8):
                dv = ebuf[drow, pl.ds(j * L, L)]
                if local_rl:
                    sl = plsc.load_gather(RL, [dv])
                else:
                    sl = tbuf[pl.ds(j * L, L)]
                sv = ebuf[srow, pl.ds(j * L, L)]
                m = sl < bound
                if soff is not None:
                    sl = sl + soff
                plsc.store_compressed(fslot.at[pl.ds(fcr, L)], sl, mask=m)
                plsc.store_compressed(fsrc.at[pl.ds(fcr, L)], sv, mask=m)
                fcr = fcr + jnp.sum(m.astype(i32))

            @pl.when(fcr >= F)
            def _fire1():
                @pl.when(fk >= 1)
                def _():
                    s_wait()
                fifo_to_bufs_and_fire(table)

            @pl.when(fcr >= 2 * F)
            def _fire2():
                s_wait()
                fifo_to_bufs_and_fire(table)
            nfired = (fcr >= F).astype(i32) + (fcr >= 2 * F).astype(i32)
            return fcr - F * nfired, fk + nfired

        def body(g, carry):
            fc, fk = carry
            return body_inner(g, fc, fk)

        nch_t = nch + jnp.where(s < nextra, 1, 0)
        fc, fk = lax.fori_loop(0, nch_t, body,
                               (jnp.int32(0), jnp.int32(0)))

        @pl.when(fc > 0)
        def _drain():
            @pl.when(fk >= 1)
            def _():
                s_wait()
            for j in range(F // L):
                idxv = j * L + lanes
                m2 = idxv < fc
                tr = trash_base + (idxv & 63)
                sbufF[pl.ds(j * L, L)] = jnp.where(
                    m2, fslot[pl.ds(j * L, L)], tr)
                gbuf[pl.ds(j * L, L)] = jnp.where(
                    m2, fsrc[pl.ds(j * L, L)], 0)
            fire(table)

        @pl.when(fk + (fc > 0).astype(i32) > 0)
        def _final_s_wait():
            s_wait()

    @pl.when(c == 0)
    def _sc0_passes():
        edge_pass(ui_ei, 0, 1, ia, True, RSTGU, TRASH_U, None, TRASH_U,
                  s * (NCH_UI * 128), NCH_UI, 4)
        edge_pass(uu_ei, 0, 1, ua, True, RSTGU, TRASH_U, OFF_UU,
                  OFF_UU + TRASH_U, s * (NCH_UU0 * 128), NCH_UU0, 4)

    @pl.when(c == 1)
    def _sc1_passes():
        edge_pass(ui_ei, 1, 0, ua, True, RSTGI, TRASH_I, None, TRASH_I,
                  s * (NCH_UI * 128), NCH_UI, 4)
        edge_pass(uu_ei, 0, 1, ua, False, RSTGU, TRASH_U, OFF_UU2,
                  OFF_UU2 + TRASH_U, UU1_BASE + s * (NCH_UU1 * 128),
                  NCH_UU1, 0)

    plsc.subcore_barrier()

    r64 = rows.at[pl.ds(0, 64)]
    s64 = tbuf.at[pl.ds(0, 64)]

    def hbm_gather(src_hbm, o, table, out):
        pltpu.sync_copy(src_hbm.at[pl.ds(o, 64)], bidx)
        pltpu.async_copy(table.at[bidx], r64, semA).wait()
        pltpu.sync_copy(r64, out.at[pl.ds(o, 64)])

    def acc_gather(rstg, o, off, out):
        pltpu.sync_copy(rstg.at[bidx], s64)
        if off is not None:
            add_off(tbuf, off, 64)
        pltpu.sync_copy(ACC.at[s64], r64)
        pltpu.sync_copy(r64, out.at[pl.ds(o, 64)])

    @pl.when(c == 0)
    def _final_user_side():
        def fin(k, _):
            o = s * 256 + k * 64
            hbm_gather(bu, o, ua, o_uego)
            acc_gather(RSTGU, o, None, o_uui)
            add_off(tbuf, OFF_UU, 64)
            pltpu.sync_copy(ACC.at[s64], r64)
            pltpu.sync_copy(r64, o_uuu1.at[pl.ds(o, 64)])
            return 0
        lax.fori_loop(0, 4, fin, 0)

    @pl.when(c == 1)
    def _final_item_side():
        def fin(k, _):
            o = s * 256 + k * 64
            pltpu.sync_copy(bu.at[pl.ds(o, 64)], bidx)
            acc_gather(RSTGU, o, OFF_UU2, o_uuu2)
            hbm_gather(bp, o, ia, o_ip)
            acc_gather(RSTGI, o, None, o_isp)
            hbm_gather(bn, o, ia, o_in)
            acc_gather(RSTGI, o, None, o_isn)
            return 0
        lax.fori_loop(0, 4, fin, 0)


_sc_call = functools.partial(
    pl.kernel,
    out_type=[jax.ShapeDtypeStruct((B, DW), jnp.float32)] * 8,
    mesh=_sc_mesh(),
    compiler_params=pltpu.CompilerParams(
        needs_layout_passes=False, use_tc_tiling_on_sc=False),
    scratch_types=[
        pltpu.VMEM_SHARED((ACC_ROWS, DW), jnp.float32),
        pltpu.VMEM_SHARED((RSZ,), jnp.int32),
        pltpu.VMEM_SHARED((RSZ,), jnp.int32),
        pltpu.VMEM((RSZ,), jnp.int32),
        pltpu.VMEM((F, DW), jnp.float32),
        pltpu.VMEM((2, 128), jnp.int32),
        pltpu.VMEM((128,), jnp.int32),
        pltpu.VMEM((256,), jnp.int32),
        pltpu.VMEM((256,), jnp.int32),
        pltpu.VMEM((F,), jnp.int32),
        pltpu.VMEM((F,), jnp.int32),
        pltpu.VMEM((64,), jnp.int32),
        pltpu.SemaphoreType.DMA,
        pltpu.SemaphoreType.DMA,
    ],
)(_sc_body)


def _tc_body(ue, uui, du, uu1, uu2, duu1, duu2, ipr, inr, isp, dp, isn, dn,
             W0, W1, W2, bm, out):
    f32 = jnp.float32
    ue_ = ue[...]
    un = uui[...] / jnp.maximum(du[...], 1.0)
    uu = (uu1[...] + uu2[...]) / jnp.maximum(duu1[...] + duu2[...], 1.0)
    h = (
        jnp.dot(ue_, W0[...], preferred_element_type=f32)
        + jnp.dot(un, W1[...], preferred_element_type=f32)
        + jnp.dot(uu, W2[...], preferred_element_type=f32)
        + bm[...]
    )
    u = jnp.tanh(h)
    ipr_ = ipr[...]
    inr_ = inr[...]
    p = 0.5 * (isp[...] / jnp.maximum(dp[...], 1.0)) + 0.5 * ipr_
    n = 0.5 * (isn[...] / jnp.maximum(dn[...], 1.0)) + 0.5 * inr_
    diff = jnp.sum(u * n, axis=-1) - jnp.sum(u * p, axis=-1)
    cf = jnp.mean(jnp.maximum(diff, 0.0) + jnp.log1p(jnp.exp(-jnp.abs(diff))))
    reg = 0.5 * jnp.mean(
        jnp.sum(ue_ * ue_, axis=-1)
        + jnp.sum(ipr_ * ipr_, axis=-1)
        + jnp.sum(inr_ * inr_, axis=-1)
    )
    out[...] = jnp.reshape(1.0 * cf + 1e-4 * reg, (1, 1))


def kernel(user_emb, item_emb, W_map, b_map, ui_edge_index, uu_edge_index,
           batch_user_pos_neg):
    f32 = jnp.float32
    i32 = jnp.int32

    ones_col = jnp.ones((NU, 1), f32)
    pad_cols = jnp.zeros((NU, DW - D - 1), f32)
    ua = jnp.concatenate([user_emb, ones_col, pad_cols], axis=1)
    ia = jnp.concatenate([item_emb, ones_col, pad_cols], axis=1)

    ui_ei = ui_edge_index.astype(i32)
    uu_ei = uu_edge_index.astype(i32)

    bu = batch_user_pos_neg[:, 0].astype(i32)
    bp = batch_user_pos_neg[:, 1].astype(i32)
    bn = batch_user_pos_neg[:, 2].astype(i32)

    pos = jnp.arange(2 * B, dtype=i32)
    ar = jnp.arange(RSZ, dtype=i32)
    initu = TRASH_U + (ar & 63)
    initi = TRASH_I + (ar & 63)
    zrows = jnp.zeros((776, DW), f32)

    (uego_a, uui_a, uuu1_a, uuu2_a, ip_a, in_a, isp_a, isn_a) = _sc_call(
        ua, ia, ui_ei, uu_ei, bu, bp, bn, pos, initu, initi, zrows)

    def split(a):
        return a[:, :D], a[:, D:D + 1]

    ue_, _ = split(uego_a)
    uui_, du = split(uui_a)
    uu1_, duu1 = split(uuu1_a)
    uu2_, duu2 = split(uuu2_a)
    ipr_, _ = split(ip_a)
    inr_, _ = split(in_a)
    isp_, dp = split(isp_a)
    isn_, dn = split(isn_a)

    W0 = W_map[:D]
    W1 = W_map[D:2 * D]
    W2 = W_map[2 * D:]
    bm = b_map.reshape(1, D)

    out = pl.pallas_call(
        _tc_body,
        out_shape=jax.ShapeDtypeStruct((1, 1), f32),
    )(ue_, uui_, du, uu1_, uu2_, duu1, duu2, ipr_, inr_, isp_, dp, isn_, dn,
      W0, W1, W2, bm)
    return out[0, 0]
